# baseline (speedup 1.0000x reference)
import functools

import jax
import jax.numpy as jnp
from jax import lax
from jax.experimental import pallas as pl
from jax.experimental.pallas import tpu as pltpu
from jax.experimental.pallas import tpu_sc as plsc

EMB_DIM = 64
BATCH, SEQ = 16384, 50
NUM_CORES = 2
NUM_SUBCORES = 16
NW = NUM_CORES * NUM_SUBCORES
CHUNK = 128
NCB = BATCH // CHUNK
CB_PER_W = NCB // NW
RT = EMB_DIM // 8
JB = 2
NB = 3
NBATCH = SEQ * (CB_PER_W // JB)
PITCH = 129
TROWS = JB * RT * 8


def _make_gather():
    mesh = plsc.VectorSubcoreMesh(core_axis_name="c", subcore_axis_name="s")

    @functools.partial(
        pl.kernel,
        mesh=mesh,
        out_type=jax.ShapeDtypeStruct((SEQ, RT, NCB, 8, CHUNK), jnp.float32),
        scratch_types=[
            pltpu.VMEM((7, CB_PER_W, 8, CHUNK), jnp.int32),
            pltpu.VMEM((NB, JB, CHUNK, EMB_DIM), jnp.float32),
            pltpu.VMEM((NB, TROWS, PITCH), jnp.float32),
            pltpu.SemaphoreType.DMA((NB,)),
            pltpu.SemaphoreType.DMA((NB,)),
        ],
        compiler_params=pltpu.CompilerParams(use_tc_tiling_on_sc=False,
                                             needs_layout_passes=False),
    )
    def gather_kernel(idx_hbm, table_hbm, out_hbm, idx_v, rows_v, tbuf,
                      gsem, wsem):
        wid = lax.axis_index("s") * NUM_CORES + lax.axis_index("c")
        lane = jax.lax.iota(jnp.int32, 16)
        rowid = [[jnp.full((16,), j * 64 + g * 16, jnp.int32) + lane
                  for g in range(4)] for j in range(JB)]

        def fire_gathers(t, bb):
            s = t // 2
            jbase = (t % 2) * JB
            for j in range(JB):
                pltpu.async_copy(
                    table_hbm.at[idx_v.at[s // 8, jbase + j, s % 8]],
                    rows_v.at[bb, j], gsem.at[bb])

        def wait_gathers(bb):
            for j in range(JB):
                pltpu.make_async_copy(table_hbm.at[idx_v.at[0, 0, 0]],
                                      rows_v.at[bb, j], gsem.at[bb]).wait()

        def wait_writes(bb):
            for j in range(JB):
                for r in range(RT):
                    pltpu.make_async_copy(
                        tbuf.at[bb, pl.ds(0, 8), pl.ds(0, CHUNK)],
                        out_hbm.at[0, r, 0], wsem.at[bb]).wait()

        def transpose(bb):
            def tbody(lq, carry):
                for lu in range(4):
                    ll = lq * 4 + lu
                    l_full = jnp.full((16,), 0, jnp.int32) + ll
                    for j in range(JB):
                        src = rows_v.at[bb, j]
                        for g in range(4):
                            v = src[ll, pl.ds(g * 16, 16)]
                            plsc.store_scatter(tbuf.at[bb],
                                               [rowid[j][g], l_full], v)
                return carry

            lax.fori_loop(0, CHUNK // 4, tbody, 0, unroll=False)

        def fire_writes(t, bb):
            s = t // 2
            cb0 = wid * CB_PER_W + (t % 2) * JB
            for j in range(JB):
                for r in range(RT):
                    pltpu.async_copy(
                        tbuf.at[bb, pl.ds(j * 64 + r * 8, 8), pl.ds(0, CHUNK)],
                        out_hbm.at[s, r, cb0 + j], wsem.at[bb])

        pltpu.sync_copy(idx_hbm.at[:, pl.ds(wid * CB_PER_W, CB_PER_W)], idx_v)
        fire_gathers(0, 0)
        fire_gathers(1, 1)

        def body(i, carry):
            for u in range(NB):
                t = i * NB + u

                @pl.when(t <= NBATCH - 1)
                def _step():
                    @pl.when(t + 2 <= NBATCH - 1)
                    def _pref():
                        fire_gathers(t + 2, (u + 2) % NB)

                    wait_gathers(u)

                    @pl.when(t >= NB)
                    def _drain():
                        wait_writes(u)

                    transpose(u)
                    fire_writes(t, u)
            return carry

        lax.fori_loop(0, (NBATCH + NB - 1) // NB, body, 0, unroll=False)
        for bb in range(NB):
            wait_writes(bb)

    return gather_kernel


_gather = _make_gather()


def kernel(input, weight):
    iv = jnp.pad(input.T, ((0, 56 - SEQ), (0, 0)))
    iv = iv.reshape(7, 8, NCB, CHUNK).transpose(0, 2, 1, 3)
    out5 = _gather(iv, weight)
    return out5.transpose(2, 4, 0, 1, 3).reshape(BATCH, SEQ, EMB_DIM)


# speedup vs baseline: 1.0071x; 1.0071x over previous
import functools

import jax
import jax.numpy as jnp
from jax import lax
from jax.experimental import pallas as pl
from jax.experimental.pallas import tpu as pltpu
from jax.experimental.pallas import tpu_sc as plsc

EMB_DIM = 64
BATCH, SEQ = 16384, 50
NUM_CORES = 2
NUM_SUBCORES = 16
NW = NUM_CORES * NUM_SUBCORES
CHUNK = 128
NCB = BATCH // CHUNK
CB_PER_W = NCB // NW
RT = EMB_DIM // 8
JB = 2
NB = 3
NBATCH = SEQ * (CB_PER_W // JB)
PITCH = 129
TROWS = JB * RT * 8


def _make_gather():
    mesh = plsc.VectorSubcoreMesh(core_axis_name="c", subcore_axis_name="s")

    @functools.partial(
        pl.kernel,
        mesh=mesh,
        out_type=jax.ShapeDtypeStruct((SEQ, RT, NCB, 8, CHUNK), jnp.float32),
        scratch_types=[
            pltpu.VMEM((7, CB_PER_W, 8, CHUNK), jnp.int32),
            pltpu.VMEM((NB, JB, CHUNK, EMB_DIM), jnp.float32),
            pltpu.VMEM((NB, JB, RT, 8, PITCH), jnp.float32),
            pltpu.SemaphoreType.DMA((NB,)),
            pltpu.SemaphoreType.DMA((NB,)),
        ],
        compiler_params=pltpu.CompilerParams(use_tc_tiling_on_sc=False,
                                             needs_layout_passes=False),
    )
    def gather_kernel(idx_hbm, table_hbm, out_hbm, idx_v, rows_v, tbuf,
                      gsem, wsem):
        wid = lax.axis_index("s") * NUM_CORES + lax.axis_index("c")
        lane = jax.lax.iota(jnp.int32, 16)
        rid = [jnp.full((16,), 2 * g, jnp.int32) + lane // 8 for g in range(4)]
        kid = lane % 8

        def fire_gathers(t, bb):
            s = t // 2
            jbase = (t % 2) * JB
            for j in range(JB):
                pltpu.async_copy(
                    table_hbm.at[idx_v.at[s // 8, jbase + j, s % 8]],
                    rows_v.at[bb, j], gsem.at[bb])

        def wait_gathers(bb):
            for j in range(JB):
                pltpu.make_async_copy(table_hbm.at[idx_v.at[0, 0, 0]],
                                      rows_v.at[bb, j], gsem.at[bb]).wait()

        def wait_writes(bb):
            for j in range(JB):
                pltpu.make_async_copy(
                    tbuf.at[bb, j, pl.ds(0, RT), pl.ds(0, 8), pl.ds(0, CHUNK)],
                    out_hbm.at[0, pl.ds(0, RT), 0], wsem.at[bb]).wait()

        def transpose(bb):
            def tbody(lq, carry):
                for lu in range(4):
                    ll = lq * 4 + lu
                    l_full = jnp.full((16,), 0, jnp.int32) + ll
                    for j in range(JB):
                        src = rows_v.at[bb, j]
                        dst = tbuf.at[bb, j]
                        for g in range(4):
                            v = src[ll, pl.ds(g * 16, 16)]
                            plsc.store_scatter(dst, [rid[g], kid, l_full], v)
                return carry

            lax.fori_loop(0, CHUNK // 4, tbody, 0, unroll=False)

        def fire_writes(t, bb):
            s = t // 2
            cb0 = wid * CB_PER_W + (t % 2) * JB
            for j in range(JB):
                pltpu.async_copy(
                    tbuf.at[bb, j, pl.ds(0, RT), pl.ds(0, 8), pl.ds(0, CHUNK)],
                    out_hbm.at[s, pl.ds(0, RT), cb0 + j], wsem.at[bb])

        pltpu.sync_copy(idx_hbm.at[:, pl.ds(wid * CB_PER_W, CB_PER_W)], idx_v)
        fire_gathers(0, 0)
        fire_gathers(1, 1)

        def body(i, carry):
            for u in range(NB):
                t = i * NB + u

                @pl.when(t <= NBATCH - 1)
                def _step():
                    @pl.when(t + 2 <= NBATCH - 1)
                    def _pref():
                        fire_gathers(t + 2, (u + 2) % NB)

                    wait_gathers(u)

                    @pl.when(t >= NB)
                    def _drain():
                        wait_writes(u)

                    transpose(u)
                    fire_writes(t, u)
            return carry

        lax.fori_loop(0, (NBATCH + NB - 1) // NB, body, 0, unroll=False)
        for bb in range(NB):
            wait_writes(bb)

    return gather_kernel


_gather = _make_gather()


def kernel(input, weight):
    iv = jnp.pad(input.T, ((0, 56 - SEQ), (0, 0)))
    iv = iv.reshape(7, 8, NCB, CHUNK).transpose(0, 2, 1, 3)
    out5 = _gather(iv, weight)
    return out5.transpose(2, 4, 0, 1, 3).reshape(BATCH, SEQ, EMB_DIM)


# speedup vs baseline: 1.0077x; 1.0006x over previous
import functools

import jax
import jax.numpy as jnp
from jax import lax
from jax.experimental import pallas as pl
from jax.experimental.pallas import tpu as pltpu
from jax.experimental.pallas import tpu_sc as plsc

EMB_DIM = 64
BATCH, SEQ = 16384, 50
NUM_CORES = 2
NUM_SUBCORES = 16
NW = NUM_CORES * NUM_SUBCORES
CHUNK = 128
NCB = BATCH // CHUNK
CB_PER_W = NCB // NW
RT = EMB_DIM // 8
JB = 2
NB = 3
NBATCH = SEQ * (CB_PER_W // JB)
PITCH = 129
TROWS = JB * RT * 8


def _make_gather():
    mesh = plsc.VectorSubcoreMesh(core_axis_name="c", subcore_axis_name="s")

    @functools.partial(
        pl.kernel,
        mesh=mesh,
        out_type=jax.ShapeDtypeStruct((SEQ, RT, NCB, 8, CHUNK), jnp.float32),
        scratch_types=[
            pltpu.VMEM((7, CB_PER_W, 8, CHUNK), jnp.int32),
            pltpu.VMEM((NB, JB, CHUNK, EMB_DIM), jnp.float32),
            pltpu.VMEM((NB, JB, RT, 8, PITCH), jnp.float32),
            pltpu.SemaphoreType.DMA((NB,)),
            pltpu.SemaphoreType.DMA((NB,)),
        ],
        compiler_params=pltpu.CompilerParams(use_tc_tiling_on_sc=False,
                                             needs_layout_passes=False,
                                             skip_device_barrier=True),
    )
    def gather_kernel(idx_hbm, table_hbm, out_hbm, idx_v, rows_v, tbuf,
                      gsem, wsem):
        wid = lax.axis_index("s") * NUM_CORES + lax.axis_index("c")
        lane = jax.lax.iota(jnp.int32, 16)
        rid = [jnp.full((16,), 2 * g, jnp.int32) + lane // 8 for g in range(4)]
        kid = lane % 8

        def fire_gathers(t, bb):
            s = t // 2
            jbase = (t % 2) * JB
            for j in range(JB):
                pltpu.async_copy(
                    table_hbm.at[idx_v.at[s // 8, jbase + j, s % 8]],
                    rows_v.at[bb, j], gsem.at[bb])

        def wait_gathers(bb):
            for j in range(JB):
                pltpu.make_async_copy(table_hbm.at[idx_v.at[0, 0, 0]],
                                      rows_v.at[bb, j], gsem.at[bb]).wait()

        def wait_writes(bb):
            for j in range(JB):
                pltpu.make_async_copy(
                    tbuf.at[bb, j, pl.ds(0, RT), pl.ds(0, 8), pl.ds(0, CHUNK)],
                    out_hbm.at[0, pl.ds(0, RT), 0], wsem.at[bb]).wait()

        def transpose(bb):
            def tbody(lq, carry):
                for lu in range(4):
                    ll = lq * 4 + lu
                    l_full = jnp.full((16,), 0, jnp.int32) + ll
                    for j in range(JB):
                        src = rows_v.at[bb, j]
                        dst = tbuf.at[bb, j]
                        for g in range(4):
                            v = src[ll, pl.ds(g * 16, 16)]
                            plsc.store_scatter(dst, [rid[g], kid, l_full], v)
                return carry

            lax.fori_loop(0, CHUNK // 4, tbody, 0, unroll=False)

        def fire_writes(t, bb):
            s = t // 2
            cb0 = wid * CB_PER_W + (t % 2) * JB
            for j in range(JB):
                pltpu.async_copy(
                    tbuf.at[bb, j, pl.ds(0, RT), pl.ds(0, 8), pl.ds(0, CHUNK)],
                    out_hbm.at[s, pl.ds(0, RT), cb0 + j], wsem.at[bb])

        pltpu.sync_copy(idx_hbm.at[:, pl.ds(wid * CB_PER_W, CB_PER_W)], idx_v)
        fire_gathers(0, 0)
        fire_gathers(1, 1)

        def body(i, carry):
            for u in range(NB):
                t = i * NB + u

                @pl.when(t <= NBATCH - 1)
                def _step():
                    @pl.when(t + 2 <= NBATCH - 1)
                    def _pref():
                        fire_gathers(t + 2, (u + 2) % NB)

                    wait_gathers(u)

                    @pl.when(t >= NB)
                    def _drain():
                        wait_writes(u)

                    transpose(u)
                    fire_writes(t, u)
            return carry

        lax.fori_loop(0, (NBATCH + NB - 1) // NB, body, 0, unroll=False)
        for bb in range(NB):
            wait_writes(bb)

    return gather_kernel


_gather = _make_gather()


def kernel(input, weight):
    iv = jnp.pad(input.T, ((0, 56 - SEQ), (0, 0)))
    iv = iv.reshape(7, 8, NCB, CHUNK).transpose(0, 2, 1, 3)
    out5 = _gather(iv, weight)
    return out5.transpose(2, 4, 0, 1, 3).reshape(BATCH, SEQ, EMB_DIM)
